# Initial kernel scaffold; baseline (speedup 1.0000x reference)
import jax
import jax.numpy as jnp
from jax.experimental import pallas as pl


def kernel(org_x, cand_x, org_edge_index, cand_edge_index, batch, W_conv, b_conv, bn_gamma, bn_beta, W_mlp, b_mlp, W_final, b_final):
    raise NotImplementedError("write your pallas kernel here")


# speedup vs baseline: 6.8851x; 6.8851x over previous
import functools

import jax
import jax.numpy as jnp
from jax import lax
from jax.experimental import pallas as pl
from jax.experimental.pallas import tpu as pltpu
from jax.experimental.pallas import tpu_sc as plsc

N = 10000
E = 320000
H = 128
NG = 16

NWORK = 32
CH = 128
CPW = 79
EPW = CH * CPW
EP = NWORK * EPW
NACC = 10240
RPT = NACC // 16
DUMMY = N

def _deg_body(ones_hbm, dst_hbm, out_hbm, dst_v, rows_v, zero_v, acc_sh):
    cid = lax.axis_index("c")
    sid = lax.axis_index("s")
    wid = cid * 16 + sid

    def zfill(i, _):
        zero_v[i // 8, pl.ds((i % 8) * 16, 16)] = jnp.zeros((16,), jnp.float32)
        return 0
    lax.fori_loop(0, CH * H // 16, zfill, 0)
    pltpu.sync_copy(ones_hbm, rows_v)

    for b in range(RPT // CH):
        pltpu.sync_copy(zero_v, acc_sh.at[pl.ds(sid * RPT + b * CH, CH)])
    plsc.subcore_barrier()

    def chunk(ch, _):
        base = pl.multiple_of(wid * EPW + ch * CH, CH)
        pltpu.sync_copy(dst_hbm.at[pl.ds(base, CH)], dst_v)
        pltpu.sync_copy(rows_v, acc_sh.at[dst_v], add=True)
        return 0
    lax.fori_loop(0, CPW, chunk, 0)
    plsc.subcore_barrier()

    for b in range(RPT // CH):
        r = sid * RPT + b * CH
        pltpu.sync_copy(acc_sh.at[pl.ds(r, CH)],
                        out_hbm.at[pl.ds(cid * NACC + r, CH)])


def _segsum_body(y_hbm, src_hbm, dst_hbm, out_hbm,
                 src_v, dst_v, rows_v, zero_v, acc_sh, sem):
    cid = lax.axis_index("c")
    sid = lax.axis_index("s")
    wid = cid * 16 + sid

    def zfill(i, _):
        zero_v[i // 8, pl.ds((i % 8) * 16, 16)] = jnp.zeros((16,), jnp.float32)
        return 0
    lax.fori_loop(0, CH * H // 16, zfill, 0)

    for b in range(RPT // CH):
        pltpu.sync_copy(zero_v, acc_sh.at[pl.ds(sid * RPT + b * CH, CH)])
    plsc.subcore_barrier()

    def chunk(ch, _):
        base = pl.multiple_of(wid * EPW + ch * CH, CH)
        pltpu.sync_copy(src_hbm.at[pl.ds(base, CH)], src_v)
        pltpu.sync_copy(dst_hbm.at[pl.ds(base, CH)], dst_v)
        pltpu.async_copy(y_hbm.at[src_v], rows_v, sem).wait()
        pltpu.sync_copy(rows_v, acc_sh.at[dst_v], add=True)
        return 0
    lax.fori_loop(0, CPW, chunk, 0)
    plsc.subcore_barrier()

    for b in range(RPT // CH):
        r = sid * RPT + b * CH
        pltpu.sync_copy(acc_sh.at[pl.ds(r, CH)],
                        out_hbm.at[pl.ds(cid * NACC + r, CH)])


@functools.lru_cache(maxsize=None)
def _sc_kernels():
    mesh = plsc.VectorSubcoreMesh(core_axis_name="c", subcore_axis_name="s")
    deg = pl.kernel(
        _deg_body,
        out_type=jax.ShapeDtypeStruct((2 * NACC, H), jnp.float32),
        mesh=mesh,
        scratch_types=[
            pltpu.VMEM((CH,), jnp.int32),
            pltpu.VMEM((CH, H), jnp.float32),
            pltpu.VMEM((CH, H), jnp.float32),
            pltpu.VMEM_SHARED((NACC, H), jnp.float32),
        ],
    )
    segsum = pl.kernel(
        _segsum_body,
        out_type=jax.ShapeDtypeStruct((2 * NACC, H), jnp.float32),
        mesh=mesh,
        scratch_types=[
            pltpu.VMEM((CH,), jnp.int32),
            pltpu.VMEM((CH,), jnp.int32),
            pltpu.VMEM((CH, H), jnp.float32),
            pltpu.VMEM((CH, H), jnp.float32),
            pltpu.VMEM_SHARED((NACC, H), jnp.float32),
            pltpu.SemaphoreType.DMA,
        ],
    )
    return deg, segsum


_R = 2000
_GRID = (N // _R,)


def _row_spec(w=H):
    return pl.BlockSpec((_R, w), lambda i: (i, 0))


def _full_spec(shape):
    return pl.BlockSpec(shape, lambda i: tuple(0 for _ in shape))


def _prep_body(x_ref, w_ref, dega_ref, degb_ref, y_ref, dinv_ref):
    d = dega_ref[:, 0:1] + degb_ref[:, 0:1] + 1.0
    dinv = lax.rsqrt(d)
    y_ref[...] = jnp.dot(x_ref[...], w_ref[...],
                         preferred_element_type=jnp.float32) * dinv
    dinv_ref[...] = jnp.broadcast_to(dinv, (_R, H))


_prep_call = pl.pallas_call(
    _prep_body,
    grid=_GRID,
    in_specs=[_row_spec(), _full_spec((H, H)), _row_spec(), _row_spec()],
    out_specs=[_row_spec(), _row_spec()],
    out_shape=[jax.ShapeDtypeStruct((N, H), jnp.float32),
               jax.ShapeDtypeStruct((N, H), jnp.float32)],
)


def _layer_body(h_ref, y_ref, acca_ref, accb_ref, dinv_ref,
                g_ref, b_ref, bt_ref, w_ref, h_out, y_out):
    tot = acca_ref[...] + accb_ref[...] + y_ref[...]
    z = tot * dinv_ref[...] + b_ref[...]
    z = jnp.maximum(z * g_ref[...] + bt_ref[...], 0.0)
    h_new = h_ref[...] + z
    h_out[...] = h_new
    y_out[...] = jnp.dot(h_new, w_ref[...],
                         preferred_element_type=jnp.float32) * dinv_ref[...]


_layer_call = pl.pallas_call(
    _layer_body,
    grid=_GRID,
    in_specs=[_row_spec(), _row_spec(), _row_spec(), _row_spec(), _row_spec(),
              _full_spec((1, H)), _full_spec((1, H)), _full_spec((1, H)),
              _full_spec((H, H))],
    out_specs=[_row_spec(), _row_spec()],
    out_shape=[jax.ShapeDtypeStruct((N, H), jnp.float32),
               jax.ShapeDtypeStruct((N, H), jnp.float32)],
)


def _last_body(h_ref, y_ref, acca_ref, accb_ref, dinv_ref,
               g_ref, b_ref, bt_ref, h_out):
    tot = acca_ref[...] + accb_ref[...] + y_ref[...]
    z = tot * dinv_ref[...] + b_ref[...]
    z = jnp.maximum(z * g_ref[...] + bt_ref[...], 0.0)
    h_out[...] = h_ref[...] + z


_last_call = pl.pallas_call(
    _last_body,
    grid=_GRID,
    in_specs=[_row_spec(), _row_spec(), _row_spec(), _row_spec(), _row_spec(),
              _full_spec((1, H)), _full_spec((1, H)), _full_spec((1, H))],
    out_specs=_row_spec(),
    out_shape=jax.ShapeDtypeStruct((N, H), jnp.float32),
)


def _pool_body(h_ref, batch_ref, pooled_ref, cnt_ref):
    @pl.when(pl.program_id(0) == 0)
    def _():
        pooled_ref[...] = jnp.zeros((NG, H), jnp.float32)
        cnt_ref[...] = jnp.zeros((NG, H), jnp.float32)

    b = batch_ref[0, 0, :]
    oh = (b[None, :] == lax.broadcasted_iota(jnp.int32, (NG, _R), 0))
    oh = oh.astype(jnp.float32)
    pooled_ref[...] += jnp.dot(oh, h_ref[...],
                               preferred_element_type=jnp.float32)
    cnt_ref[...] += jnp.broadcast_to(jnp.sum(oh, axis=1, keepdims=True),
                                     (NG, H))


_pool_call = pl.pallas_call(
    _pool_body,
    grid=_GRID,
    in_specs=[_row_spec(),
              pl.BlockSpec((1, 1, _R), lambda i: (i, 0, 0))],
    out_specs=[pl.BlockSpec((NG, H), lambda i: (0, 0)),
               pl.BlockSpec((NG, H), lambda i: (0, 0))],
    out_shape=[jax.ShapeDtypeStruct((NG, H), jnp.float32),
               jax.ShapeDtypeStruct((NG, H), jnp.float32)],
)


def _head_body(p0, p1, p2, c0, c1, c2, wm_ref, bm_ref, wf_ref, bf_ref, o_ref):
    out = jnp.broadcast_to(bf_ref[...], (NG, H))
    for i, (p, c) in enumerate(((p0, c0), (p1, c1), (p2, c2))):
        mean = p[...] / jnp.maximum(c[...], 1.0)
        hg = jnp.dot(mean, wm_ref[i], preferred_element_type=jnp.float32)
        hg = hg + bm_ref[i]
        out = out + jnp.dot(hg, wf_ref[i], preferred_element_type=jnp.float32)
    o_ref[...] = out


_head_call = pl.pallas_call(
    _head_body,
    out_shape=jax.ShapeDtypeStruct((NG, H), jnp.float32),
)


def kernel(org_x, cand_x, org_edge_index, cand_edge_index, batch,
           W_conv, b_conv, bn_gamma, bn_beta, W_mlp, b_mlp, W_final, b_final):
    xs = (org_x, cand_x[0], cand_x[1])
    eis = (org_edge_index, cand_edge_index[0], cand_edge_index[1])
    batch3d = batch.reshape(N // _R, 1, _R)
    pad = EP - E
    ones_blk = jnp.ones((CH, H), jnp.float32)

    deg_kernel, segsum_kernel = _sc_kernels()
    pooled, cnts = [], []
    for g in range(3):
        srcp = jnp.concatenate([eis[g][0], jnp.zeros((pad,), jnp.int32)])
        dstp = jnp.concatenate([eis[g][1],
                                jnp.full((pad,), DUMMY, jnp.int32)])
        deg = deg_kernel(ones_blk, dstp)
        dega, degb = deg[:NACC], deg[NACC:]
        y, dinv = _prep_call(xs[g], W_conv[g, 0], dega, degb)
        h = xs[g]
        for l in range(3):
            acc = segsum_kernel(y, srcp, dstp)
            acca, accb = acc[:NACC], acc[NACC:]
            gam = bn_gamma[g, l].reshape(1, H)
            bia = b_conv[g, l].reshape(1, H)
            bet = bn_beta[g, l].reshape(1, H)
            if l < 2:
                h, y = _layer_call(h, y, acca, accb, dinv,
                                   gam, bia, bet, W_conv[g, l + 1])
            else:
                h = _last_call(h, y, acca, accb, dinv, gam, bia, bet)
        p, c = _pool_call(h, batch3d)
        pooled.append(p)
        cnts.append(c)

    wf = jnp.pad(W_final, ((0, 0), (0, H - 10))).reshape(3, H, H)
    bf = jnp.pad(b_final, (0, H - 10)).reshape(1, H)
    bm = b_mlp.reshape(3, 1, H)
    out = _head_call(pooled[0], pooled[1], pooled[2],
                     cnts[0], cnts[1], cnts[2], W_mlp, bm, wf, bf)
    return out[:, :10]
